# baseline (speedup 1.0000x reference)
import jax
import jax.numpy as jnp
from jax.experimental import pallas as pl
from jax.experimental.pallas import tpu as pltpu

EPS_ = 1e-5
ROWS, COLS = 65536, 512
CH = 512
NCH = ROWS // CH
DEPTH = 8
NOUTER = NCH // DEPTH


def _bn_kernel(x_hbm, m_hbm, g_hbm, b_hbm, o_hbm,
               xbuf, mbuf, obuf, gloc, bloc,
               acc_s, acc_q, acc_c, coef_c, coef_b,
               sem_rx, sem_rm, sem_w, sem_misc):

    def read_x(j, s):
        return pltpu.make_async_copy(
            x_hbm.at[pl.ds(j * CH, CH), :], xbuf.at[s], sem_rx.at[s])

    def read_m(j2, b):
        return pltpu.make_async_copy(
            m_hbm.at[pl.ds(j2 * DEPTH * CH, DEPTH * CH), :], mbuf.at[b],
            sem_rm.at[b])

    def write_o(j, s):
        return pltpu.make_async_copy(
            obuf.at[s], o_hbm.at[pl.ds(j * CH, CH), :], sem_w.at[s])

    cg = pltpu.make_async_copy(g_hbm, gloc, sem_misc.at[0])
    cb = pltpu.make_async_copy(b_hbm, bloc, sem_misc.at[1])
    cg.start()
    cb.start()

    acc_s[...] = jnp.zeros_like(acc_s)
    acc_q[...] = jnp.zeros_like(acc_q)
    acc_c[...] = jnp.zeros_like(acc_c)

    for s in range(DEPTH):
        read_x(s, s).start()
    read_m(0, 0).start()

    def p0_body(j2, carry):
        read_m(j2, j2 % 2).wait()

        @pl.when(j2 + 1 < NOUTER)
        def _():
            read_m(j2 + 1, (j2 + 1) % 2).start()

        for s in range(DEPTH):
            j = j2 * DEPTH + s
            read_x(j, s).wait()
            x = xbuf[s]
            m = mbuf[j2 % 2, pl.ds(s * CH, CH), :]
            xm = x * m
            acc_s[...] += jnp.sum(xm, axis=0, keepdims=True)
            acc_q[...] += jnp.sum(xm * xm, axis=0, keepdims=True)
            acc_c[...] += jnp.sum(m, axis=0, keepdims=True)

            @pl.when(j + DEPTH < NCH)
            def _():
                read_x(j + DEPTH, s).start()
        return carry

    jax.lax.fori_loop(0, NOUTER, p0_body, 0)

    cg.wait()
    cb.wait()
    cnt = jnp.broadcast_to(acc_c[...], (1, COLS))
    mean = acc_s[...] / cnt
    var = acc_q[...] / cnt - mean * mean
    a = jax.lax.rsqrt(var + EPS_) * gloc[...]
    coef_c[...] = a - 1.0
    coef_b[...] = bloc[...] - mean * a

    for s in range(DEPTH):
        read_x(s, s).start()
    read_m(0, 0).start()

    def p1_body(j2, carry):
        read_m(j2, j2 % 2).wait()

        @pl.when(j2 + 1 < NOUTER)
        def _():
            read_m(j2 + 1, (j2 + 1) % 2).start()

        for s in range(DEPTH):
            j = j2 * DEPTH + s
            read_x(j, s).wait()

            @pl.when(j2 > 0)
            def _():
                write_o(j - DEPTH, s).wait()

            x = xbuf[s]
            m = mbuf[j2 % 2, pl.ds(s * CH, CH), :]
            t = x * coef_c[...] + coef_b[...]
            obuf[s] = x + t * m
            write_o(j, s).start()

            @pl.when(j + DEPTH < NCH)
            def _():
                read_x(j + DEPTH, s).start()
        return carry

    jax.lax.fori_loop(0, NOUTER, p1_body, 0)

    for s in range(DEPTH):
        write_o(NCH - DEPTH + s, s).wait()


def kernel(x, mask, gamma, beta):
    m = mask.astype(jnp.float32).reshape(ROWS, 1)
    g = gamma.reshape(1, COLS)
    b = beta.reshape(1, COLS)
    out = pl.pallas_call(
        _bn_kernel,
        in_specs=[
            pl.BlockSpec(memory_space=pl.ANY),
            pl.BlockSpec(memory_space=pl.ANY),
            pl.BlockSpec(memory_space=pl.ANY),
            pl.BlockSpec(memory_space=pl.ANY),
        ],
        out_specs=pl.BlockSpec(memory_space=pl.ANY),
        out_shape=jax.ShapeDtypeStruct((ROWS, COLS), x.dtype),
        scratch_shapes=[
            pltpu.VMEM((DEPTH, CH, COLS), jnp.float32),
            pltpu.VMEM((2, DEPTH * CH, 1), jnp.float32),
            pltpu.VMEM((DEPTH, CH, COLS), jnp.float32),
            pltpu.VMEM((1, COLS), jnp.float32),
            pltpu.VMEM((1, COLS), jnp.float32),
            pltpu.VMEM((1, COLS), jnp.float32),
            pltpu.VMEM((1, COLS), jnp.float32),
            pltpu.VMEM((1, 1), jnp.float32),
            pltpu.VMEM((1, COLS), jnp.float32),
            pltpu.VMEM((1, COLS), jnp.float32),
            pltpu.SemaphoreType.DMA((DEPTH,)),
            pltpu.SemaphoreType.DMA((2,)),
            pltpu.SemaphoreType.DMA((DEPTH,)),
            pltpu.SemaphoreType.DMA((2,)),
        ],
    )(x, m, g, b)
    return out


# speedup vs baseline: 1.0069x; 1.0069x over previous
import jax
import jax.numpy as jnp
from jax.experimental import pallas as pl
from jax.experimental.pallas import tpu as pltpu

EPS_ = 1e-5
ROWS, COLS = 65536, 512
CH = 2048
NCH = ROWS // CH
RD = 8
WD = 4
MD = 4
NOUTER = NCH // RD


def _bn_kernel(x_hbm, m_hbm, g_hbm, b_hbm, o_hbm,
               xbuf, mbuf, obuf, gloc, bloc,
               acc_s, acc_q, acc_c, coef_c, coef_b,
               sem_rx, sem_rm, sem_w, sem_misc):

    def read_x(j, s):
        return pltpu.make_async_copy(
            x_hbm.at[pl.ds(j * CH, CH), :], xbuf.at[s], sem_rx.at[s])

    def read_m(j, s):
        return pltpu.make_async_copy(
            m_hbm.at[pl.ds(j * CH, CH), :], mbuf.at[s], sem_rm.at[s])

    def write_o(j, s):
        return pltpu.make_async_copy(
            obuf.at[s], o_hbm.at[pl.ds(j * CH, CH), :], sem_w.at[s])

    cg = pltpu.make_async_copy(g_hbm, gloc, sem_misc.at[0])
    cb = pltpu.make_async_copy(b_hbm, bloc, sem_misc.at[1])
    cg.start()
    cb.start()

    acc_s[...] = jnp.zeros_like(acc_s)
    acc_q[...] = jnp.zeros_like(acc_q)
    acc_c[...] = jnp.zeros_like(acc_c)

    for s in range(RD):
        read_x(s, s).start()
    for s in range(MD):
        read_m(s, s).start()

    def p0_body(j2, carry):
        for s in range(RD):
            j = j2 * RD + s
            read_x(j, s).wait()
            read_m(j, j % MD).wait()
            x = xbuf[s]
            m = mbuf[j % MD]
            xm = x * m
            acc_s[...] += jnp.sum(xm, axis=0, keepdims=True)
            acc_q[...] += jnp.sum(xm * xm, axis=0, keepdims=True)
            acc_c[...] += jnp.sum(m, axis=0, keepdims=True)

            @pl.when(j + RD < NCH)
            def _():
                read_x(j + RD, s).start()

            @pl.when(j + MD < NCH)
            def _():
                read_m(j + MD, j % MD).start()
        return carry

    jax.lax.fori_loop(0, NOUTER, p0_body, 0)

    cg.wait()
    cb.wait()
    cnt = jnp.broadcast_to(acc_c[...], (1, COLS))
    mean = acc_s[...] / cnt
    var = acc_q[...] / cnt - mean * mean
    a = jax.lax.rsqrt(var + EPS_) * gloc[...]
    coef_c[...] = a - 1.0
    coef_b[...] = bloc[...] - mean * a

    for s in range(RD):
        read_x(s, s).start()
    for s in range(MD):
        read_m(s, s).start()

    def p1_body(j2, carry):
        for s in range(RD):
            j = j2 * RD + s
            read_x(j, s).wait()
            read_m(j, j % MD).wait()

            @pl.when(j >= WD)
            def _():
                write_o(j - WD, j % WD).wait()

            x = xbuf[s]
            m = mbuf[j % MD]
            t = x * coef_c[...] + coef_b[...]
            obuf[j % WD] = x + t * m
            write_o(j, j % WD).start()

            @pl.when(j + RD < NCH)
            def _():
                read_x(j + RD, s).start()

            @pl.when(j + MD < NCH)
            def _():
                read_m(j + MD, j % MD).start()
        return carry

    jax.lax.fori_loop(0, NOUTER, p1_body, 0)

    for s in range(WD):
        write_o(NCH - WD + s, (NCH - WD + s) % WD).wait()


def kernel(x, mask, gamma, beta):
    m = mask.astype(jnp.float32).reshape(ROWS, 1)
    g = gamma.reshape(1, COLS)
    b = beta.reshape(1, COLS)
    out = pl.pallas_call(
        _bn_kernel,
        in_specs=[
            pl.BlockSpec(memory_space=pl.ANY),
            pl.BlockSpec(memory_space=pl.ANY),
            pl.BlockSpec(memory_space=pl.ANY),
            pl.BlockSpec(memory_space=pl.ANY),
        ],
        out_specs=pl.BlockSpec(memory_space=pl.ANY),
        out_shape=jax.ShapeDtypeStruct((ROWS, COLS), x.dtype),
        scratch_shapes=[
            pltpu.VMEM((RD, CH, COLS), jnp.float32),
            pltpu.VMEM((MD, CH, 1), jnp.float32),
            pltpu.VMEM((WD, CH, COLS), jnp.float32),
            pltpu.VMEM((1, COLS), jnp.float32),
            pltpu.VMEM((1, COLS), jnp.float32),
            pltpu.VMEM((1, COLS), jnp.float32),
            pltpu.VMEM((1, COLS), jnp.float32),
            pltpu.VMEM((1, 1), jnp.float32),
            pltpu.VMEM((1, COLS), jnp.float32),
            pltpu.VMEM((1, COLS), jnp.float32),
            pltpu.SemaphoreType.DMA((RD,)),
            pltpu.SemaphoreType.DMA((MD,)),
            pltpu.SemaphoreType.DMA((WD,)),
            pltpu.SemaphoreType.DMA((2,)),
        ],
    )(x, m, g, b)
    return out
